# baseline (speedup 1.0000x reference)
import functools

import jax
import jax.numpy as jnp
from jax import lax
from jax.experimental import pallas as pl
from jax.experimental.pallas import tpu as pltpu
from jax.experimental.pallas import tpu_sc as plsc


def _sc_gather(tab_a, tab_b, t):
    B = t.shape[0]
    try:
        info = plsc.get_sparse_core_info()
        NC, NS, L = info.num_cores, info.num_subcores, info.num_lanes
    except Exception:
        NC, NS, L = 2, 16, 16
    NW = NC * NS
    b_per_w = B // NW

    mesh = plsc.VectorSubcoreMesh(core_axis_name="c", subcore_axis_name="s")

    @functools.partial(
        pl.kernel,
        out_type=(
            jax.ShapeDtypeStruct((B,), jnp.float32),
            jax.ShapeDtypeStruct((B,), jnp.float32),
        ),
        mesh=mesh,
        scratch_types=[
            pltpu.VMEM((b_per_w,), jnp.int32),
            pltpu.VMEM((b_per_w,), jnp.float32),
            pltpu.VMEM((b_per_w,), jnp.float32),
            pltpu.SemaphoreType.DMA,
            pltpu.SemaphoreType.DMA,
        ],
    )
    def k(tab_a_hbm, tab_b_hbm, t_hbm, out_a_hbm, out_b_hbm,
          idx_v, va_v, vb_v, sem_a, sem_b):
        wid = lax.axis_index("s") * NC + lax.axis_index("c")
        base = wid * b_per_w
        pltpu.sync_copy(t_hbm.at[pl.ds(base, b_per_w)], idx_v)
        cp_a = pltpu.async_copy(tab_a_hbm.at[idx_v], va_v, sem_a)
        cp_b = pltpu.async_copy(tab_b_hbm.at[idx_v], vb_v, sem_b)
        cp_a.wait()
        cp_b.wait()
        pltpu.sync_copy(va_v, out_a_hbm.at[pl.ds(base, b_per_w)])
        pltpu.sync_copy(vb_v, out_b_hbm.at[pl.ds(base, b_per_w)])

    return k(tab_a, tab_b, t)


def _sc_copy(src_flat):
    N = src_flat.shape[0]
    try:
        info = plsc.get_sparse_core_info()
        NC, NS = info.num_cores, info.num_subcores
    except Exception:
        NC, NS = 2, 16
    per_w = N // (NC * NS)

    mesh = plsc.VectorSubcoreMesh(core_axis_name="c", subcore_axis_name="s")

    @functools.partial(
        pl.kernel,
        out_type=jax.ShapeDtypeStruct((N,), jnp.float32),
        mesh=mesh,
    )
    def k(src_hbm, dst_hbm):
        wid = lax.axis_index("s") * NC + lax.axis_index("c")
        base = wid * per_w
        pltpu.sync_copy(src_hbm.at[pl.ds(base, per_w)],
                        dst_hbm.at[pl.ds(base, per_w)])

    return k(src_flat)


def _blend_body(a_ref, b_ref, x_ref, n_ref, o_ref):
    o_ref[...] = a_ref[...] * x_ref[...] + b_ref[...] * n_ref[...]


def _blend(a3, b3, x3, n3, row_block):
    B, S, LN = x3.shape
    scale_spec = pl.BlockSpec((row_block, 1, 1), lambda i: (i, 0, 0))
    data_spec = pl.BlockSpec((row_block, S, LN), lambda i: (i, 0, 0))
    return pl.pallas_call(
        _blend_body,
        grid=(B // row_block,),
        in_specs=[scale_spec, scale_spec, data_spec, data_spec],
        out_specs=data_spec,
        out_shape=jax.ShapeDtypeStruct((B, S, LN), jnp.float32),
    )(a3, b3, x3, n3)


def kernel(x_0, t, noise, sqrt_alphas_cumprod, sqrt_one_minus_alphas_cumprod):
    B, L = x_0.shape[0], x_0.shape[1]
    a, b = _sc_gather(sqrt_alphas_cumprod, sqrt_one_minus_alphas_cumprod,
                      t.astype(jnp.int32))
    x3 = x_0.reshape(B, L // 128, 128)
    n3 = noise.reshape(B, L // 128, 128)
    no_flat = _sc_copy(noise.reshape(B * L))
    xt3 = _blend(a.reshape(B, 1, 1), b.reshape(B, 1, 1), x3, n3,
                 row_block=256)
    return xt3.reshape(B, L, 1), no_flat.reshape(B, L, 1)


# speedup vs baseline: 15.0189x; 15.0189x over previous
import functools

import jax
import jax.numpy as jnp
from jax import lax
from jax.experimental import pallas as pl
from jax.experimental.pallas import tpu as pltpu
from jax.experimental.pallas import tpu_sc as plsc


def _sc_gather(tab_a, tab_b, t):
    B = t.shape[0]
    try:
        info = plsc.get_sparse_core_info()
        NC, NS, L = info.num_cores, info.num_subcores, info.num_lanes
    except Exception:
        NC, NS, L = 2, 16, 16
    NW = NC * NS
    b_per_w = B // NW

    mesh = plsc.VectorSubcoreMesh(core_axis_name="c", subcore_axis_name="s")

    @functools.partial(
        pl.kernel,
        out_type=(
            jax.ShapeDtypeStruct((B,), jnp.float32),
            jax.ShapeDtypeStruct((B,), jnp.float32),
        ),
        mesh=mesh,
        scratch_types=[
            pltpu.VMEM((b_per_w,), jnp.int32),
            pltpu.VMEM((b_per_w,), jnp.float32),
            pltpu.VMEM((b_per_w,), jnp.float32),
            pltpu.SemaphoreType.DMA,
            pltpu.SemaphoreType.DMA,
        ],
    )
    def k(tab_a_hbm, tab_b_hbm, t_hbm, out_a_hbm, out_b_hbm,
          idx_v, va_v, vb_v, sem_a, sem_b):
        wid = lax.axis_index("s") * NC + lax.axis_index("c")
        base = wid * b_per_w
        pltpu.sync_copy(t_hbm.at[pl.ds(base, b_per_w)], idx_v)
        cp_a = pltpu.async_copy(tab_a_hbm.at[idx_v], va_v, sem_a)
        cp_b = pltpu.async_copy(tab_b_hbm.at[idx_v], vb_v, sem_b)
        cp_a.wait()
        cp_b.wait()
        pltpu.sync_copy(va_v, out_a_hbm.at[pl.ds(base, b_per_w)])
        pltpu.sync_copy(vb_v, out_b_hbm.at[pl.ds(base, b_per_w)])

    return k(tab_a, tab_b, t)


def _sc_copy(src_flat):
    N = src_flat.shape[0]
    try:
        info = plsc.get_sparse_core_info()
        NC, NS = info.num_cores, info.num_subcores
    except Exception:
        NC, NS = 2, 16
    per_w = N // (NC * NS)
    n_chunks = 16
    ch = per_w // n_chunks

    mesh = plsc.VectorSubcoreMesh(core_axis_name="c", subcore_axis_name="s")

    @functools.partial(
        pl.kernel,
        out_type=jax.ShapeDtypeStruct((N,), jnp.float32),
        mesh=mesh,
        scratch_types=[
            pltpu.VMEM((ch,), jnp.float32),
            pltpu.VMEM((ch,), jnp.float32),
            pltpu.SemaphoreType.DMA,
            pltpu.SemaphoreType.DMA,
            pltpu.SemaphoreType.DMA,
            pltpu.SemaphoreType.DMA,
        ],
    )
    def k(src_hbm, dst_hbm, v0, v1, si0, si1, so0, so1):
        wid = lax.axis_index("s") * NC + lax.axis_index("c")
        base = wid * per_w
        bufs, sins, souts = (v0, v1), (si0, si1), (so0, so1)
        cin = [None, None]
        cout = [None, None]
        cin[0] = pltpu.async_copy(src_hbm.at[pl.ds(base, ch)], v0, si0)
        for g in range(n_chunks):
            p = g & 1
            if g + 1 < n_chunks:
                q = (g + 1) & 1
                if cout[q] is not None:
                    cout[q].wait()
                cin[q] = pltpu.async_copy(
                    src_hbm.at[pl.ds(base + (g + 1) * ch, ch)], bufs[q], sins[q])
            cin[p].wait()
            cout[p] = pltpu.async_copy(
                bufs[p], dst_hbm.at[pl.ds(base + g * ch, ch)], souts[p])
        cout[(n_chunks - 1) & 1].wait()
        if n_chunks > 1:
            cout[(n_chunks - 2) & 1].wait()

    return k(src_flat)


def _blend_body(a_ref, b_ref, x_ref, n_ref, o_ref):
    o_ref[...] = a_ref[...] * x_ref[...] + b_ref[...] * n_ref[...]


def _blend(a3, b3, x3, n3, row_block):
    B, S, LN = x3.shape
    scale_spec = pl.BlockSpec((row_block, 1, 1), lambda i: (i, 0, 0))
    data_spec = pl.BlockSpec((row_block, S, LN), lambda i: (i, 0, 0))
    return pl.pallas_call(
        _blend_body,
        grid=(B // row_block,),
        in_specs=[scale_spec, scale_spec, data_spec, data_spec],
        out_specs=data_spec,
        out_shape=jax.ShapeDtypeStruct((B, S, LN), jnp.float32),
    )(a3, b3, x3, n3)


def kernel(x_0, t, noise, sqrt_alphas_cumprod, sqrt_one_minus_alphas_cumprod):
    B, L = x_0.shape[0], x_0.shape[1]
    a, b = _sc_gather(sqrt_alphas_cumprod, sqrt_one_minus_alphas_cumprod,
                      t.astype(jnp.int32))
    x3 = x_0.reshape(B, L // 128, 128)
    n3 = noise.reshape(B, L // 128, 128)
    no_flat = _sc_copy(noise.reshape(B * L))
    xt3 = _blend(a.reshape(B, 1, 1), b.reshape(B, 1, 1), x3, n3,
                 row_block=256)
    return xt3.reshape(B, L, 1), no_flat.reshape(B, L, 1)


# speedup vs baseline: 17.5830x; 1.1707x over previous
import functools

import jax
import jax.numpy as jnp
from jax import lax
from jax.experimental import pallas as pl
from jax.experimental.pallas import tpu as pltpu
from jax.experimental.pallas import tpu_sc as plsc


def _sc_gather(tab_a, tab_b, t):
    B = t.shape[0]
    try:
        info = plsc.get_sparse_core_info()
        NC, NS, L = info.num_cores, info.num_subcores, info.num_lanes
    except Exception:
        NC, NS, L = 2, 16, 16
    NW = NC * NS
    b_per_w = B // NW

    mesh = plsc.VectorSubcoreMesh(core_axis_name="c", subcore_axis_name="s")

    @functools.partial(
        pl.kernel,
        out_type=(
            jax.ShapeDtypeStruct((B,), jnp.float32),
            jax.ShapeDtypeStruct((B,), jnp.float32),
        ),
        mesh=mesh,
        scratch_types=[
            pltpu.VMEM((b_per_w,), jnp.int32),
            pltpu.VMEM((b_per_w,), jnp.float32),
            pltpu.VMEM((b_per_w,), jnp.float32),
            pltpu.SemaphoreType.DMA,
            pltpu.SemaphoreType.DMA,
        ],
    )
    def k(tab_a_hbm, tab_b_hbm, t_hbm, out_a_hbm, out_b_hbm,
          idx_v, va_v, vb_v, sem_a, sem_b):
        wid = lax.axis_index("s") * NC + lax.axis_index("c")
        base = wid * b_per_w
        pltpu.sync_copy(t_hbm.at[pl.ds(base, b_per_w)], idx_v)
        cp_a = pltpu.async_copy(tab_a_hbm.at[idx_v], va_v, sem_a)
        cp_b = pltpu.async_copy(tab_b_hbm.at[idx_v], vb_v, sem_b)
        cp_a.wait()
        cp_b.wait()
        pltpu.sync_copy(va_v, out_a_hbm.at[pl.ds(base, b_per_w)])
        pltpu.sync_copy(vb_v, out_b_hbm.at[pl.ds(base, b_per_w)])

    return k(tab_a, tab_b, t)


def _sc_copy(src_flat):
    N = src_flat.shape[0]
    try:
        info = plsc.get_sparse_core_info()
        NC, NS = info.num_cores, info.num_subcores
    except Exception:
        NC, NS = 2, 16
    per_w = N // (NC * NS)
    n_chunks = 16
    ch = per_w // n_chunks

    mesh = plsc.VectorSubcoreMesh(core_axis_name="c", subcore_axis_name="s")

    @functools.partial(
        pl.kernel,
        out_type=jax.ShapeDtypeStruct((N,), jnp.float32),
        mesh=mesh,
        scratch_types=[
            pltpu.VMEM((ch,), jnp.float32),
            pltpu.VMEM((ch,), jnp.float32),
            pltpu.SemaphoreType.DMA,
            pltpu.SemaphoreType.DMA,
            pltpu.SemaphoreType.DMA,
            pltpu.SemaphoreType.DMA,
        ],
    )
    def k(src_hbm, dst_hbm, v0, v1, si0, si1, so0, so1):
        wid = lax.axis_index("s") * NC + lax.axis_index("c")
        base = wid * per_w
        bufs, sins, souts = (v0, v1), (si0, si1), (so0, so1)
        cin = [None, None]
        cout = [None, None]
        cin[0] = pltpu.async_copy(src_hbm.at[pl.ds(base, ch)], v0, si0)
        for g in range(n_chunks):
            p = g & 1
            if g + 1 < n_chunks:
                q = (g + 1) & 1
                if cout[q] is not None:
                    cout[q].wait()
                cin[q] = pltpu.async_copy(
                    src_hbm.at[pl.ds(base + (g + 1) * ch, ch)], bufs[q], sins[q])
            cin[p].wait()
            cout[p] = pltpu.async_copy(
                bufs[p], dst_hbm.at[pl.ds(base + g * ch, ch)], souts[p])
        cout[(n_chunks - 1) & 1].wait()
        if n_chunks > 1:
            cout[(n_chunks - 2) & 1].wait()

    return k(src_flat)


def _blend_body(a_ref, b_ref, x_ref, n_ref, o_ref, no_ref):
    nvals = n_ref[...]
    o_ref[...] = a_ref[...] * x_ref[...] + b_ref[...] * nvals
    no_ref[...] = nvals


def _blend(a3, b3, x3, n3, row_block):
    B, S, LN = x3.shape
    scale_spec = pl.BlockSpec((row_block, 1, 1), lambda i: (i, 0, 0))
    data_spec = pl.BlockSpec((row_block, S, LN), lambda i: (i, 0, 0))
    return pl.pallas_call(
        _blend_body,
        grid=(B // row_block,),
        in_specs=[scale_spec, scale_spec, data_spec, data_spec],
        out_specs=[data_spec, data_spec],
        out_shape=[
            jax.ShapeDtypeStruct((B, S, LN), jnp.float32),
            jax.ShapeDtypeStruct((B, S, LN), jnp.float32),
        ],
    )(a3, b3, x3, n3)


def kernel(x_0, t, noise, sqrt_alphas_cumprod, sqrt_one_minus_alphas_cumprod):
    B, L = x_0.shape[0], x_0.shape[1]
    a, b = _sc_gather(sqrt_alphas_cumprod, sqrt_one_minus_alphas_cumprod,
                      t.astype(jnp.int32))
    x3 = x_0.reshape(B, L // 128, 128)
    n3 = noise.reshape(B, L // 128, 128)
    xt3, no3 = _blend(a.reshape(B, 1, 1), b.reshape(B, 1, 1), x3, n3,
                      row_block=256)
    return xt3.reshape(B, L, 1), no3.reshape(B, L, 1)
